# baseline (speedup 1.0000x reference)
import functools

import jax
import jax.numpy as jnp
from jax import lax
from jax.experimental import pallas as pl
from jax.experimental.pallas import tpu as pltpu
from jax.experimental.pallas import tpu_sc as plsc

VOCAB = 100000
EMBED_DIM = 32
BATCH = 1024
_CHUNK = 128
_STRIDE = 25088


_CB = 896


def _relayout_body(x0, x1, x2, x3, y):
    y[...] = jnp.concatenate(
        [x0[...].T, x1[...].T, x2[...].T, x3[...].T], axis=1
    )


def _relayout(tableT):
    specs = []
    for a in range(4):
        specs.append(
            pl.BlockSpec(
                (EMBED_DIM, _CB),
                (lambda a_: lambda c: (0, (_STRIDE // _CB) * a_ + c))(a),
            )
        )
    return pl.pallas_call(
        _relayout_body,
        grid=(_STRIDE // _CB,),
        in_specs=specs,
        out_specs=pl.BlockSpec((_CB, _CHUNK), lambda c: (c, 0)),
        out_shape=jax.ShapeDtypeStruct((_STRIDE, _CHUNK), jnp.float32),
    )(tableT, tableT, tableT, tableT)


_info = plsc.get_sparse_core_info()
_NC, _NS = _info.num_cores, _info.num_subcores
_NW = _NC * _NS
_B_PER_W = BATCH // _NW

_sc_mesh = plsc.VectorSubcoreMesh(core_axis_name="c", subcore_axis_name="s")


@functools.partial(
    pl.kernel,
    mesh=_sc_mesh,
    out_type=jax.ShapeDtypeStruct((BATCH, _CHUNK), jnp.float32),
    scratch_types=[
        pltpu.VMEM((_B_PER_W,), jnp.int32),
        pltpu.VMEM((_B_PER_W, _CHUNK), jnp.float32),
        pltpu.SemaphoreType.DMA,
    ],
)
def _sc_gather4(table_hbm, idxc_hbm, out_hbm, idx_v, chunks_v, sem):
    wid = lax.axis_index("s") * _NC + lax.axis_index("c")
    base = wid * _B_PER_W
    pltpu.sync_copy(idxc_hbm.at[pl.ds(base, _B_PER_W)], idx_v)
    pltpu.async_copy(table_hbm.at[idx_v], chunks_v, sem).wait()
    pltpu.sync_copy(chunks_v, out_hbm.at[pl.ds(base, _B_PER_W)])


_VT = 4096


def _proj_body4(e_ref, s_ref, w_ref, b_ref, o_ref, emb_ref):
    @pl.when(pl.program_id(0) == 0)
    def _():
        lanes = lax.broadcasted_iota(jnp.int32, (BATCH, _CHUNK), 1)
        sel = (lanes // EMBED_DIM) == s_ref[...]
        e4m = jnp.where(sel, e_ref[...], 0.0)
        emb_ref[...] = (
            e4m[:, 0:32] + e4m[:, 32:64] + e4m[:, 64:96] + e4m[:, 96:128]
        )

    o_ref[...] = (
        lax.dot_general(
            w_ref[...],
            emb_ref[...],
            (((0,), (1,)), ((), ())),
            preferred_element_type=jnp.float32,
        )
        + b_ref[...][:, None]
    )


def _project_t(chunks, s2d, Wt, b):
    grid = (pl.cdiv(VOCAB, _VT),)
    return pl.pallas_call(
        _proj_body4,
        grid=grid,
        in_specs=[
            pl.BlockSpec((BATCH, _CHUNK), lambda v: (0, 0)),
            pl.BlockSpec((BATCH, 1), lambda v: (0, 0)),
            pl.BlockSpec((EMBED_DIM, _VT), lambda v: (0, v)),
            pl.BlockSpec((_VT,), lambda v: (v,)),
        ],
        out_specs=pl.BlockSpec((_VT, BATCH), lambda v: (v, 0)),
        out_shape=jax.ShapeDtypeStruct((VOCAB, BATCH), jnp.float32),
        scratch_shapes=[pltpu.VMEM((BATCH, EMBED_DIM), jnp.float32)],
    )(chunks, s2d, Wt, b)


def kernel(input_word, emb_table, W, b):
    table4 = _relayout(emb_table.T)
    idxc = input_word % _STRIDE
    s2d = (input_word // _STRIDE).reshape(BATCH, 1)
    chunks = _sc_gather4(table4, idxc)
    out_t = _project_t(chunks, s2d, W.T, b)
    return out_t.T


# speedup vs baseline: 1.0399x; 1.0399x over previous
import functools

import jax
import jax.numpy as jnp
from jax import lax
from jax.experimental import pallas as pl
from jax.experimental.pallas import tpu as pltpu
from jax.experimental.pallas import tpu_sc as plsc

VOCAB = 100000
EMBED_DIM = 32
BATCH = 1024
_CHUNK = 128
_STRIDE = 25088


_CB = 3584


def _relayout_body(x0, x1, x2, x3, y):
    y[...] = jnp.concatenate(
        [x0[...].T, x1[...].T, x2[...].T, x3[...].T], axis=1
    )


def _relayout(tableT):
    specs = []
    for a in range(4):
        specs.append(
            pl.BlockSpec(
                (EMBED_DIM, _CB),
                (lambda a_: lambda c: (0, (_STRIDE // _CB) * a_ + c))(a),
            )
        )
    return pl.pallas_call(
        _relayout_body,
        grid=(_STRIDE // _CB,),
        in_specs=specs,
        out_specs=pl.BlockSpec((_CB, _CHUNK), lambda c: (c, 0)),
        out_shape=jax.ShapeDtypeStruct((_STRIDE, _CHUNK), jnp.float32),
    )(tableT, tableT, tableT, tableT)


_info = plsc.get_sparse_core_info()
_NC, _NS = _info.num_cores, _info.num_subcores
_NW = _NC * _NS
_B_PER_W = BATCH // _NW

_sc_mesh = plsc.VectorSubcoreMesh(core_axis_name="c", subcore_axis_name="s")


@functools.partial(
    pl.kernel,
    mesh=_sc_mesh,
    out_type=jax.ShapeDtypeStruct((BATCH, _CHUNK), jnp.float32),
    scratch_types=[
        pltpu.VMEM((_B_PER_W,), jnp.int32),
        pltpu.VMEM((_B_PER_W, _CHUNK), jnp.float32),
        pltpu.SemaphoreType.DMA,
    ],
)
def _sc_gather4(table_hbm, idxc_hbm, out_hbm, idx_v, chunks_v, sem):
    wid = lax.axis_index("s") * _NC + lax.axis_index("c")
    base = wid * _B_PER_W
    pltpu.sync_copy(idxc_hbm.at[pl.ds(base, _B_PER_W)], idx_v)
    pltpu.async_copy(table_hbm.at[idx_v], chunks_v, sem).wait()
    pltpu.sync_copy(chunks_v, out_hbm.at[pl.ds(base, _B_PER_W)])


_VT = 4096


def _proj_body4(e_ref, s_ref, w_ref, b_ref, o_ref, emb_ref):
    @pl.when(pl.program_id(0) == 0)
    def _():
        lanes = lax.broadcasted_iota(jnp.int32, (BATCH, _CHUNK), 1)
        sel = (lanes // EMBED_DIM) == s_ref[...]
        e4m = jnp.where(sel, e_ref[...], 0.0)
        emb_ref[...] = (
            e4m[:, 0:32] + e4m[:, 32:64] + e4m[:, 64:96] + e4m[:, 96:128]
        )

    o_ref[...] = (
        lax.dot_general(
            w_ref[...],
            emb_ref[...],
            (((0,), (1,)), ((), ())),
            preferred_element_type=jnp.float32,
        )
        + b_ref[...][:, None]
    )


def _project_t(chunks, s2d, Wt, b):
    grid = (pl.cdiv(VOCAB, _VT),)
    return pl.pallas_call(
        _proj_body4,
        grid=grid,
        in_specs=[
            pl.BlockSpec((BATCH, _CHUNK), lambda v: (0, 0)),
            pl.BlockSpec((BATCH, 1), lambda v: (0, 0)),
            pl.BlockSpec((EMBED_DIM, _VT), lambda v: (0, v)),
            pl.BlockSpec((_VT,), lambda v: (v,)),
        ],
        out_specs=pl.BlockSpec((_VT, BATCH), lambda v: (v, 0)),
        out_shape=jax.ShapeDtypeStruct((VOCAB, BATCH), jnp.float32),
        scratch_shapes=[pltpu.VMEM((BATCH, EMBED_DIM), jnp.float32)],
    )(chunks, s2d, Wt, b)


def kernel(input_word, emb_table, W, b):
    table4 = _relayout(emb_table.T)
    idxc = input_word % _STRIDE
    s2d = (input_word // _STRIDE).reshape(BATCH, 1)
    chunks = _sc_gather4(table4, idxc)
    out_t = _project_t(chunks, s2d, W.T, b)
    return out_t.T
